# baseline (speedup 1.0000x reference)
import functools

import jax
import jax.numpy as jnp
from jax import lax
from jax.experimental import pallas as pl
from jax.experimental.pallas import tpu as pltpu
from jax.experimental.pallas import tpu_sc as plsc


def _make_lookup(B, F, V, D, NC, NS):
    NW = NC * NS
    assert B % NW == 0
    rows_w = B // NW
    R = 16
    assert rows_w % R == 0
    n_ch = rows_w // R

    mesh = plsc.VectorSubcoreMesh(core_axis_name="c", subcore_axis_name="s")

    @functools.partial(
        pl.kernel,
        mesh=mesh,
        out_type=jax.ShapeDtypeStruct((B, F, D), jnp.float32),
        scratch_types=[
            pltpu.VMEM((R, F), jnp.int32),
            pltpu.VMEM((R, F, D), jnp.float32),
            pltpu.SemaphoreType.DMA,
            pltpu.SemaphoreType.DMA,
        ],
        compiler_params=pltpu.CompilerParams(use_tc_tiling_on_sc=False),
    )
    def lookup_kernel(x_hbm, table_hbm, out_hbm, idx_v, rows_v, gsem, osem):
        wid = lax.axis_index("s") * NC + lax.axis_index("c")
        base = wid * rows_w

        def body(g, carry):
            r0 = base + g * R
            pltpu.sync_copy(x_hbm.at[pl.ds(r0, R)], idx_v)
            for r in range(R):
                pltpu.async_copy(
                    table_hbm.at[idx_v.at[r]], rows_v.at[r], gsem
                )
            for r in range(R):
                pltpu.make_async_copy(
                    table_hbm.at[idx_v.at[r]], rows_v.at[r], gsem
                ).wait()
            pltpu.async_copy(rows_v, out_hbm.at[pl.ds(r0, R)], osem).wait()
            return carry

        lax.fori_loop(0, n_ch, body, 0)

    return lookup_kernel


def kernel(x, weight):
    B, F = x.shape
    V, D = weight.shape
    info = plsc.get_sparse_core_info()
    return _make_lookup(B, F, V, D, info.num_cores, info.num_subcores)(
        x, weight
    )
